# baseline (speedup 1.0000x reference)
import functools
import math

import jax
import jax.numpy as jnp
from jax import lax
from jax.experimental import pallas as pl
from jax.experimental.pallas import tpu as pltpu
from jax.experimental.pallas import tpu_sc as plsc

_INV_SQRT2 = 1.0 / math.sqrt(2.0)
_SCB = 8
_SLAB = 16
_CH = 2


def _tc_pool_body(x_ref, o_ref, pooled_ref, *, nb, h):
    i = pl.program_id(0)
    s = x_ref[:, 0:8]
    for t in range(1, h // 8):
        s = s + x_ref[:, 8 * t:8 * t + 8]
    pooled_ref[pl.ds(i, 1), :] = jnp.sum(s, axis=(1, 2))

    @pl.when(i == nb - 1)
    def _done():
        o_ref[...] = pooled_ref[...]


def _make_sc_pool(B, H, W, C):
    nw_per_b = H // _SLAB
    mesh = plsc.VectorSubcoreMesh(core_axis_name="c", subcore_axis_name="s")

    @functools.partial(
        pl.kernel,
        out_type=jax.ShapeDtypeStruct((nw_per_b, _SCB, C), jnp.float32),
        mesh=mesh,
        scratch_types=[
            pltpu.VMEM((_CH, W, C), jnp.float32),
            pltpu.VMEM((_CH, W, C), jnp.float32),
            pltpu.VMEM((C,), jnp.float32),
            pltpu.SemaphoreType.DMA,
            pltpu.SemaphoreType.DMA,
        ],
    )
    def sc_pool(x_hbm, out_hbm, buf0, buf1, stage, s0, s1):
        nc = 2
        wid = lax.axis_index("s") * nc + lax.axis_index("c")
        q = wid % _SCB
        j = wid // _SCB
        b = (B - _SCB) + q
        h0 = j * _SLAB
        bufs = [buf0, buf1]
        sems = [s0, s1]
        nchunk = _SLAB // _CH

        def start(kc, jj):
            pltpu.make_async_copy(
                x_hbm.at[b, pl.ds(h0 + kc * _CH, _CH)],
                bufs[jj], sems[jj]).start()

        def wait(jj):
            pltpu.make_async_copy(
                x_hbm.at[b, pl.ds(h0, _CH)], bufs[jj], sems[jj]).wait()

        start(0, 0)
        ngr = C // 16

        accs = [jnp.zeros((16,), jnp.float32) for _ in range(ngr)]
        for kc in range(nchunk):
            jj = kc % 2
            wait(jj)
            if kc + 1 < nchunk:
                start(kc + 1, (kc + 1) % 2)
            buf = bufs[jj]

            def body(p, accs):
                a = p // W
                ww = p % W
                return tuple(
                    accs[g] + buf[a, ww, pl.ds(g * 16, 16)]
                    for g in range(ngr)
                )

            accs = lax.fori_loop(0, _CH * W, body, tuple(accs))

        for g in range(ngr):
            stage[pl.ds(g * 16, 16)] = accs[g]
        pltpu.sync_copy(stage, out_hbm.at[j, q])

    return sc_pool


def _mlp_body(ptc_ref, parts_ref, w1_ref, b1_ref, w2_ref, b2_ref, o_ref,
              *, nslab, inv_hw):
    psc = parts_ref[0]
    for j in range(1, nslab):
        psc = psc + parts_ref[j]
    p = jnp.concatenate([ptc_ref[...], psc], axis=0) * inv_hw
    hpre = jnp.dot(p, w1_ref[...],
                   preferred_element_type=jnp.float32,
                   precision=jax.lax.Precision.HIGHEST) + b1_ref[...]
    hact = 0.5 * hpre * (1.0 + jax.lax.erf(hpre * _INV_SQRT2))
    logits = jnp.dot(hact, w2_ref[...],
                     preferred_element_type=jnp.float32,
                     precision=jax.lax.Precision.HIGHEST) + b2_ref[...]
    mx = jnp.max(logits, axis=-1, keepdims=True)
    e = jnp.exp(logits - mx)
    o_ref[...] = e / jnp.sum(e, axis=-1, keepdims=True)


@jax.jit
def kernel(img_emb, W1, b1, W2, b2):
    B, C, H, W = img_emb.shape
    HID = W1.shape[1]
    E = W2.shape[1]
    inv_hw = 1.0 / (H * W)
    NTC = B - _SCB

    xt = jnp.transpose(img_emb, (0, 2, 3, 1))

    pooled_tc = pl.pallas_call(
        functools.partial(_tc_pool_body, nb=NTC, h=H),
        grid=(NTC,),
        in_specs=[pl.BlockSpec((1, H, W, C), lambda i: (i, 0, 0, 0))],
        out_specs=pl.BlockSpec((NTC, C), lambda i: (0, 0)),
        out_shape=jax.ShapeDtypeStruct((NTC, C), jnp.float32),
        scratch_shapes=[pltpu.VMEM((NTC, C), jnp.float32)],
    )(xt)

    parts = _make_sc_pool(B, H, W, C)(xt)

    out = pl.pallas_call(
        functools.partial(_mlp_body, nslab=H // _SLAB, inv_hw=inv_hw),
        in_specs=[
            pl.BlockSpec((NTC, C), lambda: (0, 0)),
            pl.BlockSpec((H // _SLAB, _SCB, C), lambda: (0, 0, 0)),
            pl.BlockSpec((C, HID), lambda: (0, 0)),
            pl.BlockSpec((1, HID), lambda: (0, 0)),
            pl.BlockSpec((HID, E), lambda: (0, 0)),
            pl.BlockSpec((1, E), lambda: (0, 0)),
        ],
        out_specs=pl.BlockSpec((B, E), lambda: (0, 0)),
        out_shape=jax.ShapeDtypeStruct((B, E), jnp.float32),
    )(pooled_tc, parts, W1, b1.reshape(1, -1), W2, b2.reshape(1, -1))
    return out


# speedup vs baseline: 1.3203x; 1.3203x over previous
import functools
import math

import jax
import jax.numpy as jnp
from jax.experimental import pallas as pl
from jax.experimental.pallas import tpu as pltpu

_INV_SQRT2 = 1.0 / math.sqrt(2.0)


def _body(xa_ref, xb_ref, w1_ref, b1_ref, w2_ref, b2_ref, o_ref,
          pooled_ref, *, nb, hblk, inv_hw):
    i = pl.program_id(0)
    s = xa_ref[:, 0:8] + xb_ref[:, 0:8]
    for t in range(1, hblk // 8):
        s = s + xa_ref[:, 8 * t:8 * t + 8]
        s = s + xb_ref[:, 8 * t:8 * t + 8]
    pooled_ref[pl.ds(i, 1), :] = jnp.sum(s, axis=(1, 2))

    @pl.when(i == nb - 1)
    def _finish():
        p = pooled_ref[...] * inv_hw
        hpre = jnp.dot(p, w1_ref[...],
                       preferred_element_type=jnp.float32,
                       precision=jax.lax.Precision.HIGHEST) + b1_ref[...]
        hact = 0.5 * hpre * (1.0 + jax.lax.erf(hpre * _INV_SQRT2))
        logits = jnp.dot(hact, w2_ref[...],
                         preferred_element_type=jnp.float32,
                         precision=jax.lax.Precision.HIGHEST) + b2_ref[...]
        mx = jnp.max(logits, axis=-1, keepdims=True)
        e = jnp.exp(logits - mx)
        o_ref[...] = e / jnp.sum(e, axis=-1, keepdims=True)


@jax.jit
def kernel(img_emb, W1, b1, W2, b2):
    B, C, H, W = img_emb.shape
    HID = W1.shape[1]
    E = W2.shape[1]
    inv_hw = 1.0 / (H * W)

    xt = jnp.transpose(img_emb, (0, 2, 3, 1))

    HBLK = H // 2
    out = pl.pallas_call(
        functools.partial(_body, nb=B, hblk=HBLK, inv_hw=inv_hw),
        grid=(B,),
        in_specs=[
            pl.BlockSpec((1, HBLK, W, C), lambda i: (i, 0, 0, 0)),
            pl.BlockSpec((1, HBLK, W, C), lambda i: (i, 1, 0, 0)),
            pl.BlockSpec((C, HID), lambda i: (0, 0)),
            pl.BlockSpec((1, HID), lambda i: (0, 0)),
            pl.BlockSpec((HID, E), lambda i: (0, 0)),
            pl.BlockSpec((1, E), lambda i: (0, 0)),
        ],
        out_specs=pl.BlockSpec((B, E), lambda i: (0, 0)),
        out_shape=jax.ShapeDtypeStruct((B, E), jnp.float32),
        scratch_shapes=[pltpu.VMEM((B, C), jnp.float32)],
    )(xt, xt, W1, b1.reshape(1, -1), W2, b2.reshape(1, -1))
    return out


# speedup vs baseline: 1.3264x; 1.0046x over previous
import functools
import math

import jax
import jax.numpy as jnp
from jax.experimental import pallas as pl
from jax.experimental.pallas import tpu as pltpu

_INV_SQRT2 = 1.0 / math.sqrt(2.0)


def _body(xa_ref, xb_ref, w1_ref, b1_ref, w2_ref, b2_ref, o_ref,
          pooled_ref, *, nb, hblk, inv_hw):
    i = pl.program_id(0)
    s = xa_ref[:, 0:8] + xb_ref[:, 0:8]
    for t in range(1, hblk // 8):
        s = s + xa_ref[:, 8 * t:8 * t + 8]
        s = s + xb_ref[:, 8 * t:8 * t + 8]
    pooled_ref[pl.ds(i, 1), :] = jnp.sum(s, axis=(1, 2))

    @pl.when(i == nb - 1)
    def _finish():
        p = pooled_ref[...] * inv_hw
        hpre = jnp.dot(p, w1_ref[...],
                       preferred_element_type=jnp.float32,
                       precision=jax.lax.Precision.DEFAULT) + b1_ref[...]
        hact = 0.5 * hpre * (1.0 + jax.lax.erf(hpre * _INV_SQRT2))
        logits = jnp.dot(hact, w2_ref[...],
                         preferred_element_type=jnp.float32,
                         precision=jax.lax.Precision.DEFAULT) + b2_ref[...]
        mx = jnp.max(logits, axis=-1, keepdims=True)
        e = jnp.exp(logits - mx)
        o_ref[...] = e / jnp.sum(e, axis=-1, keepdims=True)


@jax.jit
def kernel(img_emb, W1, b1, W2, b2):
    B, C, H, W = img_emb.shape
    HID = W1.shape[1]
    E = W2.shape[1]
    inv_hw = 1.0 / (H * W)

    xt = jnp.transpose(img_emb, (0, 2, 3, 1))

    HBLK = H // 2
    out = pl.pallas_call(
        functools.partial(_body, nb=B, hblk=HBLK, inv_hw=inv_hw),
        grid=(B,),
        in_specs=[
            pl.BlockSpec((1, HBLK, W, C), lambda i: (i, 0, 0, 0)),
            pl.BlockSpec((1, HBLK, W, C), lambda i: (i, 1, 0, 0)),
            pl.BlockSpec((C, HID), lambda i: (0, 0)),
            pl.BlockSpec((1, HID), lambda i: (0, 0)),
            pl.BlockSpec((HID, E), lambda i: (0, 0)),
            pl.BlockSpec((1, E), lambda i: (0, 0)),
        ],
        out_specs=pl.BlockSpec((B, E), lambda i: (0, 0)),
        out_shape=jax.ShapeDtypeStruct((B, E), jnp.float32),
        scratch_shapes=[pltpu.VMEM((B, C), jnp.float32)],
    )(xt, xt, W1, b1.reshape(1, -1), W2, b2.reshape(1, -1))
    return out
